# baseline (speedup 1.0000x reference)
import functools

import jax
import jax.numpy as jnp
from jax import lax
from jax.experimental import pallas as pl
from jax.experimental.pallas import tpu as pltpu

_CAPACITY_FACTOR = 1.25
_MIN_CAPACITY = 4


def _capacity(num_tokens, num_experts):
    cap = int(_CAPACITY_FACTOR * num_tokens / num_experts)
    cap += cap % 2
    return max(cap, _MIN_CAPACITY)


def _router_body(x_ref, out_ref, sec_ref, counts_ref, *, cap, blk):
    i = pl.program_id(0)

    @pl.when(i == 0)
    def _init():
        counts_ref[...] = jnp.zeros_like(counts_ref)

    x = x_ref[...]
    e = x.shape[-1]
    m = jnp.max(x, axis=-1, keepdims=True)
    ex = jnp.exp(x - m)
    s = jnp.sum(ex, axis=-1, keepdims=True)
    logits = ex / s

    e_iota = lax.broadcasted_iota(jnp.int32, (blk, e), 1)
    is_max = x == m
    top1 = jnp.min(jnp.where(is_max, e_iota, e), axis=-1, keepdims=True)
    mask = (e_iota == top1).astype(jnp.float32)

    r_iota = lax.broadcasted_iota(jnp.int32, (blk, blk), 0)
    c_iota = lax.broadcasted_iota(jnp.int32, (blk, blk), 1)
    ltri = (r_iota > c_iota).astype(jnp.float32)
    excl = jax.lax.dot_general(
        ltri, mask, (((1,), (0,)), ((), ())),
        preferred_element_type=jnp.float32)
    ranks = excl + counts_ref[...]
    counts_ref[...] = counts_ref[...] + jnp.sum(mask, axis=0, keepdims=True)

    keep = mask * (ranks < cap).astype(jnp.float32)
    rank_tok = jnp.sum(ranks * keep, axis=-1).astype(jnp.int32)
    w = keep * logits

    cap_iota = lax.broadcasted_iota(jnp.int32, (blk, cap), 1)
    oh = (cap_iota == rank_tok[:, None]).astype(jnp.float32)
    out = w[:, :, None] * oh[:, None, :]
    out_ref[...] = out
    sec_ref[...] = out != 0.0


def kernel(inputs):
    n, e = inputs.shape
    cap = _capacity(n, e)
    blk = 256
    grid = n // blk
    x = inputs.astype(jnp.float32)
    out, sec = pl.pallas_call(
        functools.partial(_router_body, cap=cap, blk=blk),
        grid=(grid,),
        in_specs=[pl.BlockSpec((blk, e), lambda i: (i, 0))],
        out_specs=[
            pl.BlockSpec((blk, e, cap), lambda i: (i, 0, 0)),
            pl.BlockSpec((blk, e, cap), lambda i: (i, 0, 0)),
        ],
        out_shape=[
            jax.ShapeDtypeStruct((n, e, cap), jnp.float32),
            jax.ShapeDtypeStruct((n, e, cap), jnp.bool_),
        ],
        scratch_shapes=[pltpu.VMEM((1, e), jnp.float32)],
    )(x)
    return (out.astype(inputs.dtype), sec)
